# baseline (speedup 1.0000x reference)
import jax
import jax.numpy as jnp
from jax import lax
from jax.experimental import pallas as pl
from jax.experimental.pallas import tpu as pltpu, tpu_sc as plsc

_BATCH = 4096
_IN = 2048
_SH = 256
_PW = 128
_NP = 16
_ND = 64
_BB = 512

_info = plsc.get_sparse_core_info()
_NC, _NS = _info.num_cores, _info.num_subcores
_NW = _NC * _NS
_BPW = _BATCH // _NW


def _sc_body(drug_hbm, tab_hbm, wd_out, idx_v, rows_v, sem):
    wid = lax.axis_index("s") * _NC + lax.axis_index("c")
    base = wid * _BPW
    pltpu.sync_copy(drug_hbm.at[pl.ds(base, _BPW)], idx_v)
    pltpu.async_copy(tab_hbm.at[idx_v], rows_v, sem).wait()
    pltpu.sync_copy(rows_v, wd_out.at[pl.ds(base, _BPW)])


def _sc_gather(drug_indices, table):
    mesh = plsc.VectorSubcoreMesh(core_axis_name="c", subcore_axis_name="s")
    k = pl.kernel(
        _sc_body,
        out_type=jax.ShapeDtypeStruct((_BATCH, _PW), jnp.float32),
        mesh=mesh,
        scratch_types=[
            pltpu.VMEM((_BPW,), jnp.int32),
            pltpu.VMEM((_BPW, _PW), jnp.float32),
            pltpu.SemaphoreType.DMA,
        ],
    )
    return k(drug_indices, table)


def _enc_body(x_ref, ws_ref, bs_ref, h_ref):
    xb = x_ref[...].astype(jnp.bfloat16)
    h = jnp.maximum(
        jnp.dot(xb, ws_ref[...], preferred_element_type=jnp.float32)
        + bs_ref[...], 0.0)
    h_ref[...] = h.astype(jnp.bfloat16)


def _route_body(h_ref, drug_ref, wp_ref, bp_ref, wd_ref, bdr_ref, o_ref):
    z = jnp.dot(h_ref[...], wp_ref[...],
                preferred_element_type=jnp.float32) + bp_ref[...]
    a = jnp.maximum(z, 0.0)
    drug = drug_ref[...]
    pw = drug % _NP
    oh = (drug == jax.lax.broadcasted_iota(jnp.int32, (_BB, _ND), 1)
          ).astype(jnp.float32)
    bd = jnp.dot(oh, bdr_ref[...], preferred_element_type=jnp.float32)
    colp = jax.lax.broadcasted_iota(jnp.int32, (_BB, _NP * _PW), 1) // _PW
    wd_t = jnp.concatenate([wd_ref[...]] * _NP, axis=1)
    mw = jnp.where(colp == pw, wd_t, 0.0)
    acc = jnp.sum(a * mw, axis=1, keepdims=True) + bd
    o_ref[...] = acc.reshape(_BB)


def kernel(x, drug_indices, W_shared, b_shared, W_pw, b_pw, W_drug, b_drug):
    wd = _sc_gather(drug_indices, W_drug)

    ws16 = W_shared.astype(jnp.bfloat16)
    wp16 = jnp.transpose(W_pw, (1, 0, 2)).reshape(_SH, _NP * _PW).astype(
        jnp.bfloat16)
    bp_flat = b_pw.reshape(1, _NP * _PW)
    drug2d = drug_indices.reshape(_BATCH, 1)
    bs2d = b_shared.reshape(1, _SH)
    bdr2d = b_drug.reshape(_ND, 1)

    grid = (_BATCH // _BB,)
    h = pl.pallas_call(
        _enc_body,
        grid=grid,
        in_specs=[
            pl.BlockSpec((_BB, _IN), lambda i: (i, 0)),
            pl.BlockSpec((_IN, _SH), lambda i: (0, 0)),
            pl.BlockSpec((1, _SH), lambda i: (0, 0)),
        ],
        out_specs=pl.BlockSpec((_BB, _SH), lambda i: (i, 0)),
        out_shape=jax.ShapeDtypeStruct((_BATCH, _SH), jnp.bfloat16),
    )(x, ws16, bs2d)

    out = pl.pallas_call(
        _route_body,
        grid=grid,
        in_specs=[
            pl.BlockSpec((_BB, _SH), lambda i: (i, 0)),
            pl.BlockSpec((_BB, 1), lambda i: (i, 0)),
            pl.BlockSpec((_SH, _NP * _PW), lambda i: (0, 0)),
            pl.BlockSpec((1, _NP * _PW), lambda i: (0, 0)),
            pl.BlockSpec((_BB, _PW), lambda i: (i, 0)),
            pl.BlockSpec((_ND, 1), lambda i: (0, 0)),
        ],
        out_specs=pl.BlockSpec((_BB,), lambda i: (i,)),
        out_shape=jax.ShapeDtypeStruct((_BATCH,), jnp.float32),
    )(h, drug2d, wp16, bp_flat, wd, bdr2d)
    return out


# speedup vs baseline: 1.0437x; 1.0437x over previous
import jax
import jax.numpy as jnp
from jax import lax
from jax.experimental import pallas as pl
from jax.experimental.pallas import tpu as pltpu, tpu_sc as plsc

_BATCH = 4096
_IN = 2048
_SH = 256
_PW = 128
_NP = 16
_ND = 64
_BB = 512

_info = plsc.get_sparse_core_info()
_NC, _NS = 1, _info.num_subcores
_NW = _NC * _NS
_BPW = _BATCH // _NW


def _sc_body(drug_hbm, tab_hbm, wd_out, idx_v, rows_v, sem):
    wid = lax.axis_index("s") * _NC + lax.axis_index("c")
    base = wid * _BPW
    pltpu.sync_copy(drug_hbm.at[pl.ds(base, _BPW)], idx_v)
    pltpu.async_copy(tab_hbm.at[idx_v], rows_v, sem).wait()
    pltpu.sync_copy(rows_v, wd_out.at[pl.ds(base, _BPW)])


def _sc_gather(drug_indices, table):
    mesh = plsc.VectorSubcoreMesh(core_axis_name="c", subcore_axis_name="s",
                                  num_cores=_NC)
    k = pl.kernel(
        _sc_body,
        out_type=jax.ShapeDtypeStruct((_BATCH, _PW), jnp.float32),
        mesh=mesh,
        scratch_types=[
            pltpu.VMEM((_BPW,), jnp.int32),
            pltpu.VMEM((_BPW, _PW), jnp.float32),
            pltpu.SemaphoreType.DMA,
        ],
    )
    return k(drug_indices, table)


def _enc_body(x_ref, ws_ref, bs_ref, h_ref):
    xb = x_ref[...].astype(jnp.bfloat16)
    h = jnp.maximum(
        jnp.dot(xb, ws_ref[...], preferred_element_type=jnp.float32)
        + bs_ref[...], 0.0)
    h_ref[...] = h.astype(jnp.bfloat16)


def _route_body(h_ref, drug_ref, wp_ref, bp_ref, wd_ref, bdr_ref, o_ref):
    z = jnp.dot(h_ref[...], wp_ref[...],
                preferred_element_type=jnp.float32) + bp_ref[...]
    a = jnp.maximum(z, 0.0)
    drug = drug_ref[...]
    pw = drug % _NP
    oh = (drug == jax.lax.broadcasted_iota(jnp.int32, (_BB, _ND), 1)
          ).astype(jnp.float32)
    bd = jnp.dot(oh, bdr_ref[...], preferred_element_type=jnp.float32)
    colp = jax.lax.broadcasted_iota(jnp.int32, (_BB, _NP * _PW), 1) // _PW
    wd_t = jnp.concatenate([wd_ref[...]] * _NP, axis=1)
    mw = jnp.where(colp == pw, wd_t, 0.0)
    acc = jnp.sum(a * mw, axis=1, keepdims=True) + bd
    o_ref[...] = acc.reshape(_BB)


def kernel(x, drug_indices, W_shared, b_shared, W_pw, b_pw, W_drug, b_drug):
    wd = _sc_gather(drug_indices, W_drug)

    ws16 = W_shared.astype(jnp.bfloat16)
    wp16 = jnp.transpose(W_pw, (1, 0, 2)).reshape(_SH, _NP * _PW).astype(
        jnp.bfloat16)
    bp_flat = b_pw.reshape(1, _NP * _PW)
    drug2d = drug_indices.reshape(_BATCH, 1)
    bs2d = b_shared.reshape(1, _SH)
    bdr2d = b_drug.reshape(_ND, 1)

    grid = (_BATCH // _BB,)
    h = pl.pallas_call(
        _enc_body,
        grid=grid,
        in_specs=[
            pl.BlockSpec((_BB, _IN), lambda i: (i, 0)),
            pl.BlockSpec((_IN, _SH), lambda i: (0, 0)),
            pl.BlockSpec((1, _SH), lambda i: (0, 0)),
        ],
        out_specs=pl.BlockSpec((_BB, _SH), lambda i: (i, 0)),
        out_shape=jax.ShapeDtypeStruct((_BATCH, _SH), jnp.bfloat16),
    )(x, ws16, bs2d)

    out = pl.pallas_call(
        _route_body,
        grid=grid,
        in_specs=[
            pl.BlockSpec((_BB, _SH), lambda i: (i, 0)),
            pl.BlockSpec((_BB, 1), lambda i: (i, 0)),
            pl.BlockSpec((_SH, _NP * _PW), lambda i: (0, 0)),
            pl.BlockSpec((1, _NP * _PW), lambda i: (0, 0)),
            pl.BlockSpec((_BB, _PW), lambda i: (i, 0)),
            pl.BlockSpec((_ND, 1), lambda i: (0, 0)),
        ],
        out_specs=pl.BlockSpec((_BB,), lambda i: (i,)),
        out_shape=jax.ShapeDtypeStruct((_BATCH,), jnp.float32),
    )(h, drug2d, wp16, bp_flat, wd, bdr2d)
    return out


# speedup vs baseline: 1.0839x; 1.0385x over previous
import jax
import jax.numpy as jnp
from jax import lax
from jax.experimental import pallas as pl
from jax.experimental.pallas import tpu as pltpu, tpu_sc as plsc

_BATCH = 4096
_IN = 2048
_SH = 256
_PW = 128
_NP = 16
_ND = 64
_BB = 1024

_info = plsc.get_sparse_core_info()
_NC, _NS = 1, _info.num_subcores
_NW = _NC * _NS
_BPW = _BATCH // _NW


def _sc_body(drug_hbm, tab_hbm, wd_out, idx_v, rows_v, sem):
    wid = lax.axis_index("s") * _NC + lax.axis_index("c")
    base = wid * _BPW
    pltpu.sync_copy(drug_hbm.at[pl.ds(base, _BPW)], idx_v)
    pltpu.async_copy(tab_hbm.at[idx_v], rows_v, sem).wait()
    pltpu.sync_copy(rows_v, wd_out.at[pl.ds(base, _BPW)])


def _sc_gather(drug_indices, table):
    mesh = plsc.VectorSubcoreMesh(core_axis_name="c", subcore_axis_name="s",
                                  num_cores=_NC)
    k = pl.kernel(
        _sc_body,
        out_type=jax.ShapeDtypeStruct((_BATCH, _PW), jnp.float32),
        mesh=mesh,
        scratch_types=[
            pltpu.VMEM((_BPW,), jnp.int32),
            pltpu.VMEM((_BPW, _PW), jnp.float32),
            pltpu.SemaphoreType.DMA,
        ],
    )
    return k(drug_indices, table)


def _enc_body(x_ref, ws_ref, bs_ref, h_ref):
    xb = x_ref[...].astype(jnp.bfloat16)
    h = jnp.maximum(
        jnp.dot(xb, ws_ref[...], preferred_element_type=jnp.float32)
        + bs_ref[...], 0.0)
    h_ref[...] = h.astype(jnp.bfloat16)


def _route_body(h_ref, drug_ref, wp_ref, bp_ref, wd_ref, bdr_ref, o_ref):
    z = jnp.dot(h_ref[...], wp_ref[...],
                preferred_element_type=jnp.float32) + bp_ref[...]
    a = jnp.maximum(z, 0.0)
    drug = drug_ref[...]
    pw = drug % _NP
    oh = (drug == jax.lax.broadcasted_iota(jnp.int32, (_BB, _ND), 1)
          ).astype(jnp.float32)
    bd = jnp.dot(oh, bdr_ref[...], preferred_element_type=jnp.float32)
    colp = jax.lax.broadcasted_iota(jnp.int32, (_BB, _NP * _PW), 1) // _PW
    wd_t = jnp.concatenate([wd_ref[...]] * _NP, axis=1)
    mw = jnp.where(colp == pw, wd_t, 0.0)
    acc = jnp.sum(a * mw, axis=1, keepdims=True) + bd
    o_ref[...] = acc.reshape(_BB)


def kernel(x, drug_indices, W_shared, b_shared, W_pw, b_pw, W_drug, b_drug):
    wd = _sc_gather(drug_indices, W_drug)

    ws16 = W_shared.astype(jnp.bfloat16)
    wp16 = jnp.transpose(W_pw, (1, 0, 2)).reshape(_SH, _NP * _PW).astype(
        jnp.bfloat16)
    bp_flat = b_pw.reshape(1, _NP * _PW)
    drug2d = drug_indices.reshape(_BATCH, 1)
    bs2d = b_shared.reshape(1, _SH)
    bdr2d = b_drug.reshape(_ND, 1)

    grid = (_BATCH // _BB,)
    h = pl.pallas_call(
        _enc_body,
        grid=grid,
        in_specs=[
            pl.BlockSpec((_BB, _IN), lambda i: (i, 0)),
            pl.BlockSpec((_IN, _SH), lambda i: (0, 0)),
            pl.BlockSpec((1, _SH), lambda i: (0, 0)),
        ],
        out_specs=pl.BlockSpec((_BB, _SH), lambda i: (i, 0)),
        out_shape=jax.ShapeDtypeStruct((_BATCH, _SH), jnp.bfloat16),
    )(x, ws16, bs2d)

    out = pl.pallas_call(
        _route_body,
        grid=grid,
        in_specs=[
            pl.BlockSpec((_BB, _SH), lambda i: (i, 0)),
            pl.BlockSpec((_BB, 1), lambda i: (i, 0)),
            pl.BlockSpec((_SH, _NP * _PW), lambda i: (0, 0)),
            pl.BlockSpec((1, _NP * _PW), lambda i: (0, 0)),
            pl.BlockSpec((_BB, _PW), lambda i: (i, 0)),
            pl.BlockSpec((_ND, 1), lambda i: (0, 0)),
        ],
        out_specs=pl.BlockSpec((_BB,), lambda i: (i,)),
        out_shape=jax.ShapeDtypeStruct((_BATCH,), jnp.float32),
    )(h, drug2d, wp16, bp_flat, wd, bdr2d)
    return out
